# Initial kernel scaffold; baseline (speedup 1.0000x reference)
import jax
import jax.numpy as jnp
from jax.experimental import pallas as pl


def kernel(pooled_output, centroids, labels, delta):
    raise NotImplementedError("write your pallas kernel here")


# speedup vs baseline: 2.8234x; 2.8234x over previous
import dataclasses
import functools

import jax
import jax.numpy as jnp
from jax import lax
from jax.experimental import pallas as pl
from jax.experimental.pallas import tpu as pltpu
from jax.experimental.pallas import tpu_sc as plsc

_SAFE1 = 0.1
_SAFE2 = 0.5

_ROWS = 8192
_D = 128
_NCENT = 1000
_NW = 32
_RPW = _ROWS // _NW
_LANES = 16


def _sc_distances(pooled3, labels2, centroids, delta):
  mesh = plsc.VectorSubcoreMesh(core_axis_name="c", subcore_axis_name="s")
  f32 = jnp.float32
  cp = pltpu.CompilerParams()
  if "needs_layout_passes" in pltpu.CompilerParams.__dataclass_fields__:
    cp = dataclasses.replace(cp, needs_layout_passes=False)

  @functools.partial(
      pl.kernel,
      compiler_params=cp,
      out_type=(
          jax.ShapeDtypeStruct((_ROWS,), f32),
          jax.ShapeDtypeStruct((_ROWS,), f32),
          jax.ShapeDtypeStruct((_ROWS,), f32),
      ),
      mesh=mesh,
      scratch_types=[
          pltpu.VMEM((2, 128), jnp.int32),
          pltpu.VMEM((_RPW, _D), f32),
          pltpu.VMEM((_RPW, 2, _D), f32),
          pltpu.VMEM((_NCENT,), f32),
          pltpu.VMEM((_RPW,), f32),
          pltpu.VMEM((_RPW,), f32),
          pltpu.VMEM((_RPW,), f32),
          pltpu.SemaphoreType.DMA,
          pltpu.SemaphoreType.DMA,
          pltpu.SemaphoreType.DMA,
          pltpu.SemaphoreType.DMA,
      ],
  )
  def sc_kernel(pooled_hbm, labels_hbm, cent_hbm, delta_hbm,
                sx_hbm, sn_hbm, dg_hbm,
                lbl_v, c_v, po_v, dtab_v, sx_v, sn_v, dg_v,
                sem_a, sem_b, sem_c, sem_g):
    wid = lax.axis_index("s") * 2 + lax.axis_index("c")
    base = wid * _RPW

    cp_lbl = pltpu.async_copy(labels_hbm.at[pl.ds(wid * 2, 2)], lbl_v, sem_a)
    cp_po = pltpu.async_copy(pooled_hbm.at[pl.ds(base, _RPW)], po_v, sem_b)
    cp_dt = pltpu.async_copy(delta_hbm, dtab_v, sem_c)
    cp_lbl.wait()

    cp_g0 = pltpu.async_copy(
        cent_hbm.at[lbl_v.at[0]], c_v.at[pl.ds(0, 128)], sem_g)
    cp_g1 = pltpu.async_copy(
        cent_hbm.at[lbl_v.at[1]], c_v.at[pl.ds(128, 128)], sem_g)

    cp_dt.wait()
    for t in range(_RPW // _LANES):
      idx = lbl_v[t // 8, pl.ds((t % 8) * _LANES, _LANES)]
      dg_v[pl.ds(t * _LANES, _LANES)] = plsc.load_gather(dtab_v, [idx])

    cp_po.wait()
    cp_g0.wait()
    cp_g1.wait()

    lane = lax.iota(jnp.int32, _LANES)

    @pl.loop(0, _RPW // _LANES)
    def _(t):
      rsx = jnp.zeros((_LANES,), f32)
      rsn = jnp.zeros((_LANES,), f32)
      for l in range(_LANES):
        r = t * _LANES + l
        accx = jnp.zeros((_LANES,), f32)
        accn = jnp.zeros((_LANES,), f32)
        for k in range(_D // _LANES):
          sl = pl.ds(k * _LANES, _LANES)
          c = c_v[r, sl]
          x = po_v[r, 0, sl]
          n = po_v[r, 1, sl]
          tx = x - c
          tn = n - c
          accx = accx + tx * tx
          accn = accn + tn * tn
        rsx = jnp.where(lane == l, jnp.sum(accx), rsx)
        rsn = jnp.where(lane == l, jnp.sum(accn), rsn)
      sx_v[pl.ds(t * _LANES, _LANES)] = rsx
      sn_v[pl.ds(t * _LANES, _LANES)] = rsn

    o1 = pltpu.async_copy(sx_v, sx_hbm.at[pl.ds(base, _RPW)], sem_a)
    o2 = pltpu.async_copy(sn_v, sn_hbm.at[pl.ds(base, _RPW)], sem_b)
    o3 = pltpu.async_copy(dg_v, dg_hbm.at[pl.ds(base, _RPW)], sem_c)
    o1.wait()
    o2.wait()
    o3.wait()

  return sc_kernel(pooled3, labels2, centroids, delta)


def _tc_finish(sx, sn, dg, delta):
  f32 = jnp.float32

  def body(sx_ref, sn_ref, dg_ref, delta_ref, loss_ref, dsp_ref):
    euc = jnp.sqrt(sx_ref[...])
    neu = jnp.sqrt(sn_ref[...])
    d = jax.nn.softplus(dg_ref[...])
    pos = jnp.maximum(euc - d, 0.0)
    neg = jnp.maximum(d - euc, 0.0)
    npos = jnp.maximum(neu - (d + _SAFE2), 0.0)
    nneg = jnp.maximum((d - neu) + _SAFE1, 0.0)
    total = (jnp.sum(pos) + jnp.sum(neg)) + (jnp.sum(npos) + jnp.sum(nneg))
    loss_ref[...] = jnp.broadcast_to(total * (1.0 / _ROWS), (1, 1))
    dsp_ref[...] = jax.nn.softplus(delta_ref[...])

  return pl.pallas_call(
      body,
      out_shape=(
          jax.ShapeDtypeStruct((1, 1), f32),
          jax.ShapeDtypeStruct((8, 125), f32),
      ),
  )(sx.reshape(64, 128), sn.reshape(64, 128), dg.reshape(64, 128),
    delta.reshape(8, 125))


def kernel(pooled_output, centroids, labels, delta):
  pooled3 = pooled_output.reshape(_ROWS, 2, _D)
  labels2 = labels.reshape(64, 128)
  sx, sn, dg = _sc_distances(pooled3, labels2, centroids, delta)
  loss2, dsp2 = _tc_finish(sx, sn, dg, delta)
  return loss2[0, 0], dsp2.reshape(_NCENT)
